# baseline (speedup 1.0000x reference)
import functools

import jax
import jax.numpy as jnp
import numpy as np
from jax.experimental import pallas as pl
from jax.experimental.pallas import tpu as pltpu

_SIGMA_X = 0.1
_SIGMA_Y = 0.1
_C_LOGSIG = np.float32(2.0 * np.log(_SIGMA_Y))
_C_LOG2PI = np.float32(np.log(2.0 * np.pi))

_NT = (((1,), (1,)), ((), ()))


def _fused_body(xt_ref, nt_ref, lwt_ref, f_ref, g_ref, y_ref,
                xtt_ref, lwn_ref, m_ref, s_ref, mean_ref, cov_ref,
                m_acc, s_acc, sx_acc, cov_acc, xstash, lwstash,
                *, num_blocks, block):
    p = pl.program_id(0)
    i = pl.program_id(1)
    f32 = jnp.float32

    @pl.when((p == 0) & (i == 0))
    def _init0():
        m_acc[0, 0] = jnp.float32(-jnp.inf)
        s_acc[0, 0] = jnp.float32(0.0)
        sx_acc[...] = jnp.zeros_like(sx_acc)

    @pl.when(p == 0)
    def _phase0():
        xrow = jax.lax.dot_general(xt_ref[...], f_ref[...], _NT,
                                   preferred_element_type=f32)
        xrow = xrow + jnp.float32(_SIGMA_X) * nt_ref[...]
        xtt_ref[...] = xrow
        xb = xrow.T
        xstash[:, pl.ds(i * block, block)] = xb

        ym = jnp.dot(g_ref[...], xb, preferred_element_type=f32)
        dd = (y_ref[...] - ym) / jnp.float32(_SIGMA_Y)
        terms = dd * dd + _C_LOGSIG + _C_LOG2PI
        lp = -0.5 * jnp.sum(terms, axis=0, keepdims=True)

        lwn = lwt_ref[...] + lp
        lwn_ref[...] = lwn
        lwstash[:, pl.ds(i * block, block)] = lwn

        bm = jnp.max(lwn)
        m_old = m_acc[0, 0]
        m_new = jnp.maximum(m_old, bm)
        alpha = jnp.exp(m_old - m_new)
        w = jnp.exp(lwn - m_new)
        s_acc[0, 0] = s_acc[0, 0] * alpha + jnp.sum(w)
        sx_acc[...] = (sx_acc[...] * alpha
                       + jnp.sum(w * xb, axis=1, keepdims=True))
        m_acc[0, 0] = m_new

        @pl.when(i == num_blocks - 1)
        def _finish0():
            m_ref[0, 0] = m_new
            s_tot = s_acc[0, 0]
            s_ref[0, 0] = s_tot
            mean_ref[...] = sx_acc[...] / s_tot

    @pl.when(p == 1)
    def _phase1():
        @pl.when(i == 0)
        def _init1():
            cov_acc[...] = jnp.zeros_like(cov_acc)

        xb = xstash[:, pl.ds(i * block, block)]
        lwn = lwstash[:, pl.ds(i * block, block)]
        w = jnp.exp(lwn - m_acc[0, 0])
        mean = sx_acc[...] / s_acc[0, 0]
        xc = xb - mean
        wxc = xc * w
        cov_acc[...] += jax.lax.dot_general(wxc, xc, _NT,
                                            preferred_element_type=f32)

        @pl.when(i == num_blocks - 1)
        def _finish1():
            cov_ref[...] = cov_acc[...] / s_acc[0, 0]


def _pipeline(x_base, lw_base, y_T, noise, F, G):
    n, xdim = x_base.shape
    ydim = y_T.shape[0]
    block = 8192 if n % 8192 == 0 else n
    num_blocks = n // block
    last = num_blocks - 1

    lwt = lw_base.reshape(1, n)
    ycol = y_T.reshape(ydim, 1)

    def pin(io):
        return lambda p, i: (0, io(i) * (1 - p) + last * p)

    f32 = jnp.float32
    xtt, lwn, m_max, s_sum, mean_col, cov = pl.pallas_call(
        functools.partial(_fused_body, num_blocks=num_blocks, block=block),
        grid=(2, num_blocks),
        in_specs=[
            pl.BlockSpec((block, xdim), lambda p, i: (i * (1 - p) + (num_blocks - 1) * p, 0)),
            pl.BlockSpec((block, xdim), lambda p, i: (i * (1 - p) + (num_blocks - 1) * p, 0)),
            pl.BlockSpec((1, block), pin(lambda i: i)),
            pl.BlockSpec((xdim, xdim), lambda p, i: (0, 0)),
            pl.BlockSpec((ydim, xdim), lambda p, i: (0, 0)),
            pl.BlockSpec((ydim, 1), lambda p, i: (0, 0)),
        ],
        out_specs=[
            pl.BlockSpec((block, xdim), lambda p, i: (i * (1 - p) + (num_blocks - 1) * p, 0)),
            pl.BlockSpec((1, block), pin(lambda i: i)),
            pl.BlockSpec(memory_space=pltpu.SMEM),
            pl.BlockSpec(memory_space=pltpu.SMEM),
            pl.BlockSpec((xdim, 1), lambda p, i: (0, 0)),
            pl.BlockSpec((xdim, xdim), lambda p, i: (0, 0)),
        ],
        out_shape=[
            jax.ShapeDtypeStruct((n, xdim), f32),
            jax.ShapeDtypeStruct((1, n), f32),
            jax.ShapeDtypeStruct((1, 1), f32),
            jax.ShapeDtypeStruct((1, 1), f32),
            jax.ShapeDtypeStruct((xdim, 1), f32),
            jax.ShapeDtypeStruct((xdim, xdim), f32),
        ],
        scratch_shapes=[
            pltpu.SMEM((1, 1), f32),
            pltpu.SMEM((1, 1), f32),
            pltpu.VMEM((xdim, 1), f32),
            pltpu.VMEM((xdim, xdim), f32),
            pltpu.VMEM((xdim, n), f32),
            pltpu.VMEM((1, n), f32),
        ],
    )(x_base, noise, lwt, F, G, ycol)

    x_T = xtt
    log_w_new = lwn.reshape(n, 1)
    x_t_mean = mean_col.reshape(xdim)
    return x_T, log_w_new, x_t_mean, cov


def kernel(x_Tm1, log_w, y_T, noise, F, G):
    n = x_Tm1.shape[0]
    lw = log_w[:, 0]
    log_ess = (2.0 * jax.scipy.special.logsumexp(lw)
               - jax.scipy.special.logsumexp(2.0 * lw))
    do_resample = log_ess <= np.log(n / 2.0)

    def _resampled(_):
        key = jax.random.key(42)
        ancestors = jax.random.categorical(key, lw, shape=(n,))
        x_r = jnp.take(x_Tm1, ancestors, axis=0)
        lw_r = jnp.full_like(log_w, -np.log(n))
        return _pipeline(x_r, lw_r, y_T, noise, F, G)

    def _plain(_):
        return _pipeline(x_Tm1, log_w, y_T, noise, F, G)

    return jax.lax.cond(do_resample, _resampled, _plain, operand=None)


# speedup vs baseline: 3.9627x; 3.9627x over previous
import functools

import jax
import jax.numpy as jnp
import numpy as np
from jax.experimental import pallas as pl
from jax.experimental.pallas import tpu as pltpu

_SIGMA_X = 0.1
_SIGMA_Y = 0.1
_C_LOGSIG = np.float32(2.0 * np.log(_SIGMA_Y))
_C_LOG2PI = np.float32(np.log(2.0 * np.pi))

_NT = (((1,), (1,)), ((), ()))


def _fused_body(xt_ref, nt_ref, lwt_ref, f_ref, g_ref, y_ref,
                xtt_ref, lwn_ref, m_ref, s_ref, mean_ref, cov_ref,
                m_acc, s_acc, sx_acc, cov_acc, xstash, lwstash,
                *, num_blocks, block):
    p = pl.program_id(0)
    i = pl.program_id(1)
    f32 = jnp.float32

    @pl.when((p == 0) & (i == 0))
    def _init0():
        m_acc[0, 0] = jnp.float32(-jnp.inf)
        s_acc[0, 0] = jnp.float32(0.0)
        sx_acc[...] = jnp.zeros_like(sx_acc)

    @pl.when(p == 0)
    def _phase0():
        xb = jnp.dot(f_ref[...], xt_ref[...], preferred_element_type=f32)
        xb = xb + jnp.float32(_SIGMA_X) * nt_ref[...]
        xtt_ref[...] = xb
        xstash[:, pl.ds(i * block, block)] = xb

        ym = jnp.dot(g_ref[...], xb, preferred_element_type=f32)
        dd = (y_ref[...] - ym) / jnp.float32(_SIGMA_Y)
        terms = dd * dd + _C_LOGSIG + _C_LOG2PI
        lp = -0.5 * jnp.sum(terms, axis=0, keepdims=True)

        lwn = lwt_ref[...] + lp
        lwn_ref[...] = lwn
        lwstash[:, pl.ds(i * block, block)] = lwn

        bm = jnp.max(lwn)
        m_old = m_acc[0, 0]
        m_new = jnp.maximum(m_old, bm)
        alpha = jnp.exp(m_old - m_new)
        w = jnp.exp(lwn - m_new)
        s_acc[0, 0] = s_acc[0, 0] * alpha + jnp.sum(w)
        sx_acc[...] = (sx_acc[...] * alpha
                       + jnp.sum(w * xb, axis=1, keepdims=True))
        m_acc[0, 0] = m_new

        @pl.when(i == num_blocks - 1)
        def _finish0():
            m_ref[0, 0] = m_new
            s_tot = s_acc[0, 0]
            s_ref[0, 0] = s_tot
            mean_ref[...] = sx_acc[...] / s_tot

    @pl.when(p == 1)
    def _phase1():
        @pl.when(i == 0)
        def _init1():
            cov_acc[...] = jnp.zeros_like(cov_acc)

        xb = xstash[:, pl.ds(i * block, block)]
        lwn = lwstash[:, pl.ds(i * block, block)]
        w = jnp.exp(lwn - m_acc[0, 0])
        mean = sx_acc[...] / s_acc[0, 0]
        xc = xb - mean
        wxc = xc * w
        cov_acc[...] += jax.lax.dot_general(wxc, xc, _NT,
                                            preferred_element_type=f32)

        @pl.when(i == num_blocks - 1)
        def _finish1():
            cov_ref[...] = cov_acc[...] / s_acc[0, 0]


def _pipeline(x_base, lw_base, y_T, noise, F, G):
    n, xdim = x_base.shape
    ydim = y_T.shape[0]
    block = 16384 if n % 16384 == 0 else n
    num_blocks = n // block
    last = num_blocks - 1

    xt = x_base.T
    nt = noise.T
    lwt = lw_base.reshape(1, n)
    ycol = y_T.reshape(ydim, 1)

    def pin(io):
        return lambda p, i: (0, io(i) * (1 - p) + last * p)

    f32 = jnp.float32
    xtt, lwn, m_max, s_sum, mean_col, cov = pl.pallas_call(
        functools.partial(_fused_body, num_blocks=num_blocks, block=block),
        grid=(2, num_blocks),
        in_specs=[
            pl.BlockSpec((xdim, block), pin(lambda i: i)),
            pl.BlockSpec((xdim, block), pin(lambda i: i)),
            pl.BlockSpec((1, block), pin(lambda i: i)),
            pl.BlockSpec((xdim, xdim), lambda p, i: (0, 0)),
            pl.BlockSpec((ydim, xdim), lambda p, i: (0, 0)),
            pl.BlockSpec((ydim, 1), lambda p, i: (0, 0)),
        ],
        out_specs=[
            pl.BlockSpec((xdim, block), pin(lambda i: i)),
            pl.BlockSpec((1, block), pin(lambda i: i)),
            pl.BlockSpec(memory_space=pltpu.SMEM),
            pl.BlockSpec(memory_space=pltpu.SMEM),
            pl.BlockSpec((xdim, 1), lambda p, i: (0, 0)),
            pl.BlockSpec((xdim, xdim), lambda p, i: (0, 0)),
        ],
        out_shape=[
            jax.ShapeDtypeStruct((xdim, n), f32),
            jax.ShapeDtypeStruct((1, n), f32),
            jax.ShapeDtypeStruct((1, 1), f32),
            jax.ShapeDtypeStruct((1, 1), f32),
            jax.ShapeDtypeStruct((xdim, 1), f32),
            jax.ShapeDtypeStruct((xdim, xdim), f32),
        ],
        scratch_shapes=[
            pltpu.SMEM((1, 1), f32),
            pltpu.SMEM((1, 1), f32),
            pltpu.VMEM((xdim, 1), f32),
            pltpu.VMEM((xdim, xdim), f32),
            pltpu.VMEM((xdim, n), f32),
            pltpu.VMEM((1, n), f32),
        ],
    )(xt, nt, lwt, F, G, ycol)

    x_T = xtt.T
    log_w_new = lwn.reshape(n, 1)
    x_t_mean = mean_col.reshape(xdim)
    return x_T, log_w_new, x_t_mean, cov


def kernel(x_Tm1, log_w, y_T, noise, F, G):
    n = x_Tm1.shape[0]
    lw = log_w[:, 0]
    log_ess = (2.0 * jax.scipy.special.logsumexp(lw)
               - jax.scipy.special.logsumexp(2.0 * lw))
    do_resample = log_ess <= np.log(n / 2.0)

    def _resampled(_):
        key = jax.random.key(42)
        ancestors = jax.random.categorical(key, lw, shape=(n,))
        x_r = jnp.take(x_Tm1, ancestors, axis=0)
        lw_r = jnp.full_like(log_w, -np.log(n))
        return _pipeline(x_r, lw_r, y_T, noise, F, G)

    def _plain(_):
        return _pipeline(x_Tm1, log_w, y_T, noise, F, G)

    return jax.lax.cond(do_resample, _resampled, _plain, operand=None)
